# baseline (speedup 1.0000x reference)
import jax
import jax.numpy as jnp
from jax import lax
from jax.experimental import pallas as pl
from jax.experimental.pallas import tpu as pltpu
from jax.experimental.pallas import tpu_sc as plsc

L = 4
N = 2 * L + 1
NF = N * N

_PAIRS = [[(m1, mu + L - m1) for m1 in range(max(0, mu - L), min(N, mu + L + 1))]
          for mu in range(N)]

B_IN = 20000
NUM_WORKERS = 32
CHUNK = 160
GROUPS = CHUNK // 16
NSLICES = 2


def _tree_sum(vals):
    while len(vals) > 1:
        nxt = [a + b for a, b in zip(vals[::2], vals[1::2])]
        if len(vals) % 2:
            nxt.append(vals[-1])
        vals = nxt
    return vals[0]


def _make_body(b_slice):
    rows_per_worker = -(-b_slice // (NUM_WORKERS * CHUNK)) * CHUNK
    nchunks = rows_per_worker // CHUNK

    def _body(x1_hbm, x2_hbm, out_hbm, x1_v, x2_v, out_v):
        nc = 2
        wid = lax.axis_index("s") * nc + lax.axis_index("c")
        base = wid * rows_per_worker
        lane81 = lax.broadcasted_iota(jnp.int32, (16,), 0) * NF

        def chunk_body(ci, carry):
            cb = jnp.minimum(base + ci * CHUNK, b_slice - CHUNK) * NF
            pltpu.sync_copy(x1_hbm.at[pl.ds(cb, CHUNK * NF)], x1_v)
            pltpu.sync_copy(x2_hbm.at[pl.ds(cb, CHUNK * NF)], x2_v)

            def group_body(g, c2):
                row81 = g * (16 * NF) + lane81
                for mu in range(N):
                    acc = [None] * N
                    for (m1, m2) in _PAIRS[mu]:
                        a = [plsc.load_gather(x1_v, [row81 + (m1 * N + j)])
                             for j in range(N)]
                        b = [plsc.load_gather(x2_v, [row81 + (m2 * N + j)])
                             for j in range(N)]
                        for mup in range(N):
                            part = _tree_sum(
                                [a[p] * b[q] for (p, q) in _PAIRS[mup]])
                            acc[mup] = (part if acc[mup] is None
                                        else acc[mup] + part)
                    for mup in range(N):
                        plsc.store_scatter(out_v, [row81 + (mu * N + mup)],
                                           acc[mup])
                return c2

            lax.fori_loop(0, GROUPS, group_body, 0)
            pltpu.sync_copy(out_v, out_hbm.at[pl.ds(cb, CHUNK * NF)])
            return carry

        lax.fori_loop(0, nchunks, chunk_body, 0)

    return _body


def _make_run(b_slice):
    mesh = plsc.VectorSubcoreMesh(core_axis_name="c", subcore_axis_name="s")
    return pl.kernel(
        _make_body(b_slice),
        out_type=jax.ShapeDtypeStruct((b_slice * NF,), jnp.float32),
        mesh=mesh,
        compiler_params=pltpu.CompilerParams(
            needs_layout_passes=False,
            disable_bounds_checks=True,
        ),
        scratch_types=[
            pltpu.VMEM((CHUNK * NF,), jnp.float32),
            pltpu.VMEM((CHUNK * NF,), jnp.float32),
            pltpu.VMEM((CHUNK * NF,), jnp.float32),
        ],
    )


@jax.jit
def kernel(X1, X2):
    b = X1.shape[0]
    h = b // NSLICES
    x1f = X1.reshape(b * NF)
    x2f = X2.reshape(b * NF)
    run = _make_run(h)
    outs = []
    for s in range(NSLICES):
        o = run(x1f[s * h * NF:(s + 1) * h * NF],
                x2f[s * h * NF:(s + 1) * h * NF])
        outs.append(o.reshape(h, N, N))
    return jnp.concatenate(outs, axis=0)


# speedup vs baseline: 1.1147x; 1.1147x over previous
import jax
import jax.numpy as jnp
from jax import lax
from jax.experimental import pallas as pl
from jax.experimental.pallas import tpu as pltpu
from jax.experimental.pallas import tpu_sc as plsc

L = 4
N = 2 * L + 1
NF = N * N

_PAIRS = [[(m1, mu + L - m1) for m1 in range(max(0, mu - L), min(N, mu + L + 1))]
          for mu in range(N)]

B_IN = 20000
NUM_WORKERS = 32
CHUNK = 160
GROUPS = CHUNK // 16
NSLICES = 4


def _tree_sum(vals):
    while len(vals) > 1:
        nxt = [a + b for a, b in zip(vals[::2], vals[1::2])]
        if len(vals) % 2:
            nxt.append(vals[-1])
        vals = nxt
    return vals[0]


def _make_body(b_slice):
    rows_per_worker = -(-b_slice // (NUM_WORKERS * CHUNK)) * CHUNK
    nchunks = rows_per_worker // CHUNK

    def _body(x1_hbm, x2_hbm, out_hbm, x1_v, x2_v, out_v):
        nc = 2
        wid = lax.axis_index("s") * nc + lax.axis_index("c")
        base = wid * rows_per_worker
        lane81 = lax.broadcasted_iota(jnp.int32, (16,), 0) * NF

        def chunk_body(ci, carry):
            cb = jnp.minimum(base + ci * CHUNK, b_slice - CHUNK) * NF
            pltpu.sync_copy(x1_hbm.at[pl.ds(cb, CHUNK * NF)], x1_v)
            pltpu.sync_copy(x2_hbm.at[pl.ds(cb, CHUNK * NF)], x2_v)

            def group_body(g, c2):
                row81 = g * (16 * NF) + lane81
                for mu in range(N):
                    acc = [None] * N
                    for (m1, m2) in _PAIRS[mu]:
                        a = [plsc.load_gather(x1_v, [row81 + (m1 * N + j)])
                             for j in range(N)]
                        b = [plsc.load_gather(x2_v, [row81 + (m2 * N + j)])
                             for j in range(N)]
                        for mup in range(N):
                            part = _tree_sum(
                                [a[p] * b[q] for (p, q) in _PAIRS[mup]])
                            acc[mup] = (part if acc[mup] is None
                                        else acc[mup] + part)
                    for mup in range(N):
                        plsc.store_scatter(out_v, [row81 + (mu * N + mup)],
                                           acc[mup])
                return c2

            lax.fori_loop(0, GROUPS, group_body, 0)
            pltpu.sync_copy(out_v, out_hbm.at[pl.ds(cb, CHUNK * NF)])
            return carry

        lax.fori_loop(0, nchunks, chunk_body, 0)

    return _body


def _make_run(b_slice):
    mesh = plsc.VectorSubcoreMesh(core_axis_name="c", subcore_axis_name="s")
    return pl.kernel(
        _make_body(b_slice),
        out_type=jax.ShapeDtypeStruct((b_slice * NF,), jnp.float32),
        mesh=mesh,
        compiler_params=pltpu.CompilerParams(
            needs_layout_passes=False,
            disable_bounds_checks=True,
        ),
        scratch_types=[
            pltpu.VMEM((CHUNK * NF,), jnp.float32),
            pltpu.VMEM((CHUNK * NF,), jnp.float32),
            pltpu.VMEM((CHUNK * NF,), jnp.float32),
        ],
    )


@jax.jit
def kernel(X1, X2):
    b = X1.shape[0]
    h = b // NSLICES
    x1f = X1.reshape(b * NF)
    x2f = X2.reshape(b * NF)
    run = _make_run(h)
    outs = []
    for s in range(NSLICES):
        o = run(x1f[s * h * NF:(s + 1) * h * NF],
                x2f[s * h * NF:(s + 1) * h * NF])
        outs.append(o.reshape(h, N, N))
    return jnp.concatenate(outs, axis=0)
